# baseline (speedup 1.0000x reference)
import jax
import jax.numpy as jnp
from jax import lax
from jax.experimental import pallas as pl
from jax.experimental.pallas import tpu as pltpu
from jax.experimental.pallas import tpu_sc as plsc

V = 1000000
D = 64
B = 4096
L = 20
K = 5

NC = 2
NS = 16
NW = NC * NS
BW = B // NW
SCH = 16
NCHUNK = BW // SCH


def _sc_scores_body(tgt_g, tgt_o, ctx_g, ctx_o, neg_g, neg_o,
                    in_emb, out_emb,
                    pos_out, neg_out,
                    tgt_gi, tgt_oi, ctx_gi, ctx_oi, neg_gi, neg_oi,
                    t_rows, ctx_rows, neg_rows, t_cols,
                    pos_buf, neg_buf, sem):
    c_id = lax.axis_index("c")
    s_id = lax.axis_index("s")
    wid = s_id * NC + c_id
    base = wid * BW

    pltpu.sync_copy(tgt_g.at[pl.ds(base, BW)], tgt_gi)
    pltpu.sync_copy(tgt_o.at[pl.ds(base, BW)], tgt_oi)
    pltpu.sync_copy(ctx_g.at[pl.ds(base * L, BW * L)], ctx_gi)
    pltpu.sync_copy(ctx_o.at[pl.ds(base * L, BW * L)], ctx_oi)
    pltpu.sync_copy(neg_g.at[pl.ds(base * K, BW * K)], neg_gi)
    pltpu.sync_copy(neg_o.at[pl.ds(base * K, BW * K)], neg_oi)

    pltpu.async_copy(in_emb.at[tgt_gi], t_rows, sem).wait()

    lanes = lax.iota(jnp.int32, 16)
    row20 = lanes * L
    row5 = lanes * K

    def chunk_body(chunk, carry):
        cbase = pl.multiple_of(chunk * (SCH * L), 8)
        nbase = pl.multiple_of(chunk * (SCH * K), 8)
        h1 = pltpu.async_copy(
            out_emb.at[ctx_gi.at[pl.ds(cbase, 120)]],
            ctx_rows.at[pl.ds(0, 120)], sem)
        h2 = pltpu.async_copy(
            out_emb.at[ctx_gi.at[pl.ds(cbase + 120, 120)]],
            ctx_rows.at[pl.ds(120, 120)], sem)
        h3 = pltpu.async_copy(
            out_emb.at[ctx_gi.at[pl.ds(cbase + 240, 80)]],
            ctx_rows.at[pl.ds(240, 80)], sem)
        h4 = pltpu.async_copy(
            out_emb.at[neg_gi.at[pl.ds(nbase, SCH * K)]],
            neg_rows, sem)

        t_row = lanes + chunk * SCH
        t_off = plsc.load_gather(tgt_oi, [t_row])

        def tstage(d4, _):
            for u in range(4):
                dd = d4 * 4 + u
                v = plsc.load_gather(t_rows, [t_row, t_off + dd])
                t_cols[pl.ds(pl.multiple_of(dd * 16, 16), 16)] = v
            return 0

        lax.fori_loop(0, D // 4, tstage, 0)

        h1.wait()
        h2.wait()
        h3.wait()
        h4.wait()

        def pair_body(j, _, rows, oi, buf, rowbase, nper):
            par = plsc.load_gather(oi, [(lanes + chunk * SCH) * nper + j])
            rowv = rowbase + j

            def dblock(d8, accs):
                a0, a1 = accs
                for u in range(8):
                    dd = d8 * 8 + u
                    tv = t_cols[pl.ds(pl.multiple_of(dd * 16, 16), 16)]
                    cv = plsc.load_gather(rows, [rowv, par + dd])
                    if u % 2 == 0:
                        a0 = a0 + tv * cv
                    else:
                        a1 = a1 + tv * cv
                return (a0, a1)

            z = jnp.zeros((16,), jnp.float32)
            a0, a1 = lax.fori_loop(0, D // 8, dblock, (z, z))
            buf[pl.ds(pl.multiple_of((chunk * nper + j) * 16, 16), 16)] = a0 + a1
            return 0

        lax.fori_loop(0, L, lambda j, c: pair_body(
            j, c, ctx_rows, ctx_oi, pos_buf, row20, L), 0)
        lax.fori_loop(0, K, lambda j, c: pair_body(
            j, c, neg_rows, neg_oi, neg_buf, row5, K), 0)
        return 0

    lax.fori_loop(0, NCHUNK, chunk_body, 0)

    pltpu.sync_copy(pos_buf, pos_out.at[pl.ds(wid * (BW * L), BW * L)])
    pltpu.sync_copy(neg_buf, neg_out.at[pl.ds(wid * (BW * K), BW * K)])


_sc_scores = pl.kernel(
    _sc_scores_body,
    out_type=(
        jax.ShapeDtypeStruct((B * L,), jnp.float32),
        jax.ShapeDtypeStruct((B * K,), jnp.float32),
    ),
    mesh=plsc.VectorSubcoreMesh(
        core_axis_name="c", subcore_axis_name="s",
        num_cores=NC, num_subcores=NS),
    compiler_params=pltpu.CompilerParams(
        needs_layout_passes=False, use_tc_tiling_on_sc=True),
    scratch_types=(
        pltpu.VMEM((BW,), jnp.int32),
        pltpu.VMEM((BW,), jnp.int32),
        pltpu.VMEM((BW * L,), jnp.int32),
        pltpu.VMEM((BW * L,), jnp.int32),
        pltpu.VMEM((BW * K,), jnp.int32),
        pltpu.VMEM((BW * K,), jnp.int32),
        pltpu.VMEM((BW, 2 * D), jnp.float32),
        pltpu.VMEM((SCH * L, 2 * D), jnp.float32),
        pltpu.VMEM((SCH * K, 2 * D), jnp.float32),
        pltpu.VMEM((D * 16,), jnp.float32),
        pltpu.VMEM((BW * L,), jnp.float32),
        pltpu.VMEM((BW * K,), jnp.float32),
        pltpu.SemaphoreType.DMA,
    ),
)


def _tp_body(a_ref, oa_ref):
    xt = a_ref[...].T
    x3 = xt.reshape(2048, 2, 64)
    oa_ref[...] = jnp.concatenate([x3[:, 0, :], x3[:, 1, :]], axis=1)


_tp_table = pl.pallas_call(
    _tp_body,
    grid=(245,),
    in_specs=[pl.BlockSpec((D, 4096), lambda p: (0, p))],
    out_specs=pl.BlockSpec((2048, 2 * D), lambda p: (p, 0)),
    out_shape=jax.ShapeDtypeStruct((V // 2, 2 * D), jnp.float32),
)


def _loss_body(pos_ref, neg_ref, out_ref):
    p = pos_ref[...]
    n = neg_ref[...]
    pos_sum = jnp.sum(jnp.maximum(-p, 0.0) + jnp.log1p(jnp.exp(-jnp.abs(p))))
    neg_sum = jnp.sum(jnp.maximum(n, 0.0) + jnp.log1p(jnp.exp(-jnp.abs(n))))
    out_ref[0, 0] = pos_sum / (B * L)
    out_ref[0, 1] = neg_sum / (B * K)


def _loss(pos_scores, neg_scores):
    return pl.pallas_call(
        _loss_body,
        out_shape=jax.ShapeDtypeStruct((1, 2), jnp.float32),
        out_specs=pl.BlockSpec(memory_space=pltpu.SMEM),
    )(pos_scores, neg_scores)


@jax.jit
def kernel(target_words, context_words_list, negative_words, input_emb, output_emb):
    tw = target_words.reshape(-1).astype(jnp.int32)
    cw = context_words_list.reshape(-1).astype(jnp.int32)
    nw = negative_words.reshape(-1).astype(jnp.int32)
    in2 = _tp_table(input_emb.T)
    out2 = output_emb.reshape(V // 2, 2 * D)
    pos_scores, neg_scores = _sc_scores(
        tw >> 1, (tw & 1) * D,
        cw >> 1, (cw & 1) * D,
        nw >> 1, (nw & 1) * D,
        in2, out2)
    out = _loss(pos_scores.reshape(B * L // 128, 128),
                neg_scores.reshape(B * K // 128, 128))
    return (out[0, 0], out[0, 1])


# speedup vs baseline: 1.0687x; 1.0687x over previous
import jax
import jax.numpy as jnp
from jax import lax
from jax.experimental import pallas as pl
from jax.experimental.pallas import tpu as pltpu
from jax.experimental.pallas import tpu_sc as plsc

V = 1000000
D = 64
B = 4096
L = 20
K = 5

NC = 2
NS = 16
NW = NC * NS
BW = B // NW
SCH = 16
NCHUNK = BW // SCH


def _sc_scores_body(tgt_g, tgt_o, ctx_g, ctx_o, neg_g, neg_o,
                    in_emb, out_emb,
                    pos_out, neg_out,
                    tgt_gi, tgt_oi, ctx_gi, ctx_oi, neg_gi, neg_oi,
                    t_rows, ctx_rows, neg_rows, t_cols,
                    pos_buf, neg_buf, sem):
    c_id = lax.axis_index("c")
    s_id = lax.axis_index("s")
    wid = s_id * NC + c_id
    base = wid * BW

    pltpu.sync_copy(tgt_g.at[pl.ds(base, BW)], tgt_gi)
    pltpu.sync_copy(tgt_o.at[pl.ds(base, BW)], tgt_oi)
    pltpu.sync_copy(ctx_g.at[pl.ds(base * L, BW * L)], ctx_gi)
    pltpu.sync_copy(ctx_o.at[pl.ds(base * L, BW * L)], ctx_oi)
    pltpu.sync_copy(neg_g.at[pl.ds(base * K, BW * K)], neg_gi)
    pltpu.sync_copy(neg_o.at[pl.ds(base * K, BW * K)], neg_oi)

    pltpu.async_copy(in_emb.at[tgt_gi], t_rows, sem).wait()

    lanes = lax.iota(jnp.int32, 16)
    row20 = lanes * L
    row5 = lanes * K

    def chunk_body(chunk, carry):
        cbase = pl.multiple_of(chunk * (SCH * L), 8)
        nbase = pl.multiple_of(chunk * (SCH * K), 8)
        h1 = pltpu.async_copy(
            out_emb.at[ctx_gi.at[pl.ds(cbase, 120)]],
            ctx_rows.at[pl.ds(0, 120)], sem)
        h2 = pltpu.async_copy(
            out_emb.at[ctx_gi.at[pl.ds(cbase + 120, 120)]],
            ctx_rows.at[pl.ds(120, 120)], sem)
        h3 = pltpu.async_copy(
            out_emb.at[ctx_gi.at[pl.ds(cbase + 240, 80)]],
            ctx_rows.at[pl.ds(240, 80)], sem)
        h4 = pltpu.async_copy(
            out_emb.at[neg_gi.at[pl.ds(nbase, SCH * K)]],
            neg_rows, sem)

        t_row = lanes + chunk * SCH
        t_off = plsc.load_gather(tgt_oi, [t_row])

        def tstage(d4, _):
            for u in range(4):
                dd = d4 * 4 + u
                v = plsc.load_gather(t_rows, [t_row, t_off + dd])
                t_cols[pl.ds(pl.multiple_of(dd * 16, 16), 16)] = v
            return 0

        lax.fori_loop(0, D // 4, tstage, 0)

        h1.wait()
        h2.wait()
        h3.wait()
        h4.wait()

        def pair_body(j, _, rows, oi, buf, rowbase, nper):
            par = plsc.load_gather(oi, [(lanes + chunk * SCH) * nper + j])
            rowv = rowbase + j

            def dblock(d8, accs):
                a0, a1 = accs
                for u in range(8):
                    dd = d8 * 8 + u
                    tv = t_cols[pl.ds(pl.multiple_of(dd * 16, 16), 16)]
                    cv = plsc.load_gather(rows, [rowv, par + dd])
                    if u % 2 == 0:
                        a0 = a0 + tv * cv
                    else:
                        a1 = a1 + tv * cv
                return (a0, a1)

            z = jnp.zeros((16,), jnp.float32)
            a0, a1 = lax.fori_loop(0, D // 8, dblock, (z, z))
            buf[pl.ds(pl.multiple_of((chunk * nper + j) * 16, 16), 16)] = a0 + a1
            return 0

        lax.fori_loop(0, L, lambda j, c: pair_body(
            j, c, ctx_rows, ctx_oi, pos_buf, row20, L), 0)
        lax.fori_loop(0, K, lambda j, c: pair_body(
            j, c, neg_rows, neg_oi, neg_buf, row5, K), 0)
        return 0

    lax.fori_loop(0, NCHUNK, chunk_body, 0)

    pltpu.sync_copy(pos_buf, pos_out.at[pl.ds(wid * (BW * L), BW * L)])
    pltpu.sync_copy(neg_buf, neg_out.at[pl.ds(wid * (BW * K), BW * K)])


_sc_scores = pl.kernel(
    _sc_scores_body,
    out_type=(
        jax.ShapeDtypeStruct((B * L,), jnp.float32),
        jax.ShapeDtypeStruct((B * K,), jnp.float32),
    ),
    mesh=plsc.VectorSubcoreMesh(
        core_axis_name="c", subcore_axis_name="s",
        num_cores=NC, num_subcores=NS),
    compiler_params=pltpu.CompilerParams(
        needs_layout_passes=False, use_tc_tiling_on_sc=True),
    scratch_types=(
        pltpu.VMEM((BW,), jnp.int32),
        pltpu.VMEM((BW,), jnp.int32),
        pltpu.VMEM((BW * L,), jnp.int32),
        pltpu.VMEM((BW * L,), jnp.int32),
        pltpu.VMEM((BW * K,), jnp.int32),
        pltpu.VMEM((BW * K,), jnp.int32),
        pltpu.VMEM((BW, 2 * D), jnp.float32),
        pltpu.VMEM((SCH * L, 2 * D), jnp.float32),
        pltpu.VMEM((SCH * K, 2 * D), jnp.float32),
        pltpu.VMEM((D * 16,), jnp.float32),
        pltpu.VMEM((BW * L,), jnp.float32),
        pltpu.VMEM((BW * K,), jnp.float32),
        pltpu.SemaphoreType.DMA,
    ),
)


def _tp_body(a_ref, b_ref, oa_ref, ob_ref):
    for x_ref, o_ref in ((a_ref, oa_ref), (b_ref, ob_ref)):
        xt = x_ref[...].T
        x3 = xt.reshape(4096, 2, 64)
        o_ref[...] = jnp.concatenate([x3[:, 0, :], x3[:, 1, :]], axis=1)


_tp_tables = pl.pallas_call(
    _tp_body,
    grid=(123,),
    in_specs=[pl.BlockSpec((D, 8192), lambda p: (0, p)),
              pl.BlockSpec((D, 8192), lambda p: (0, p))],
    out_specs=[pl.BlockSpec((4096, 2 * D), lambda p: (p, 0)),
               pl.BlockSpec((4096, 2 * D), lambda p: (p, 0))],
    out_shape=[jax.ShapeDtypeStruct((V // 2, 2 * D), jnp.float32),
               jax.ShapeDtypeStruct((V // 2, 2 * D), jnp.float32)],
)


def _loss_body(pos_ref, neg_ref, out_ref):
    p = pos_ref[...]
    n = neg_ref[...]
    pos_sum = jnp.sum(jnp.maximum(-p, 0.0) + jnp.log1p(jnp.exp(-jnp.abs(p))))
    neg_sum = jnp.sum(jnp.maximum(n, 0.0) + jnp.log1p(jnp.exp(-jnp.abs(n))))
    out_ref[0, 0] = pos_sum / (B * L)
    out_ref[0, 1] = neg_sum / (B * K)


def _loss(pos_scores, neg_scores):
    return pl.pallas_call(
        _loss_body,
        out_shape=jax.ShapeDtypeStruct((1, 2), jnp.float32),
        out_specs=pl.BlockSpec(memory_space=pltpu.SMEM),
    )(pos_scores, neg_scores)


@jax.jit
def kernel(target_words, context_words_list, negative_words, input_emb, output_emb):
    tw = target_words.reshape(-1).astype(jnp.int32)
    cw = context_words_list.reshape(-1).astype(jnp.int32)
    nw = negative_words.reshape(-1).astype(jnp.int32)
    in2, out2 = _tp_tables(input_emb.T, output_emb.T)
    pos_scores, neg_scores = _sc_scores(
        tw >> 1, (tw & 1) * D,
        cw >> 1, (cw & 1) * D,
        nw >> 1, (nw & 1) * D,
        in2, out2)
    out = _loss(pos_scores.reshape(B * L // 128, 128),
                neg_scores.reshape(B * K // 128, 128))
    return (out[0, 0], out[0, 1])
